# speedup vs baseline: 4.5301x; 4.5301x over previous
import jax
import jax.numpy as jnp
from jax import lax
from jax.experimental import pallas as pl

_TOPK = 64
_BR = 8


def _tc_body(x_ref, o_ref):
    x = x_ref[...]
    r, n = x.shape
    k = _TOPK

    s = lax.bitcast_convert_type(x, jnp.int32)
    y = jnp.where(s < 0, s ^ jnp.int32(0x7FFFFFFF), s)

    lo0 = jnp.full((r, 1), -jnp.int32(2**31 - 1) - 1, jnp.int32)
    hi0 = jnp.full((r, 1), jnp.int32(2**31 - 1), jnp.int32)

    def bs_iter(_, carry):
        lo, hi = carry
        mid = (lo & hi) + ((lo ^ hi) >> 1)
        cnt = jnp.sum((y >= mid).astype(jnp.int32), axis=1, keepdims=True)
        ge = cnt >= k
        return jnp.where(ge, mid, lo), jnp.where(ge, hi, mid)

    lo, _ = lax.fori_loop(0, 32, bs_iter, (lo0, hi0))
    t = lo

    eq = y == t
    cge = jnp.sum((y >= t).astype(jnp.int32), axis=1, keepdims=True)
    cgt = jnp.sum((y > t).astype(jnp.int32), axis=1, keepdims=True)
    need = k - cgt

    iota = lax.broadcasted_iota(jnp.int32, (r, n), 1)

    def tie_search():
        ilo0 = jnp.zeros((r, 1), jnp.int32)
        ihi0 = jnp.full((r, 1), n, jnp.int32)

        def it(_, carry):
            ilo, ihi = carry
            mid = (ilo + ihi) >> 1
            c = jnp.sum((eq & (iota < mid)).astype(jnp.int32), axis=1,
                        keepdims=True)
            ok = c >= need
            return jnp.where(ok, ilo, mid), jnp.where(ok, mid, ihi)

        _, ihi = lax.fori_loop(0, 15, it, (ilo0, ihi0))
        return ihi

    cut = lax.cond(jnp.max(cge) > k, tie_search,
                   lambda: jnp.full((r, 1), n, jnp.int32))

    mask = (y > t) | (eq & (iota < cut))
    e = jnp.exp(-jnp.abs(x))
    sig = jnp.where(x >= 0, 1.0 / (1.0 + e), e / (1.0 + e))
    o_ref[...] = jnp.where(mask, sig, jnp.float32(0.0))


def kernel(x):
    r, n = x.shape
    return pl.pallas_call(
        _tc_body,
        grid=(r // _BR,),
        in_specs=[pl.BlockSpec((_BR, n), lambda i: (i, 0))],
        out_specs=pl.BlockSpec((_BR, n), lambda i: (i, 0)),
        out_shape=jax.ShapeDtypeStruct((r, n), x.dtype),
    )(x)
